# baseline (speedup 1.0000x reference)
import functools

import jax
import jax.numpy as jnp
from jax import lax
from jax.experimental import pallas as pl
from jax.experimental.pallas import tpu as pltpu
from jax.experimental.pallas import tpu_sc as plsc

B = 4096
S = 50
D = 64
SP = 56
DP = 128


@functools.cache
def _build_lookup():
    info = plsc.get_sparse_core_info()
    nc, ns = info.num_cores, info.num_subcores
    nw = nc * ns
    rows_per_w = B // nw
    nbuf = 6
    lead = 3

    mesh = plsc.VectorSubcoreMesh(core_axis_name="c", subcore_axis_name="s")

    def body(idx_hbm, table_hbm, out_hbm, idx_v, rows_v, gsem, ssem):
        wid = lax.axis_index("s") * nc + lax.axis_index("c")
        row0 = wid * rows_per_w
        pltpu.sync_copy(idx_hbm.at[pl.ds(row0, rows_per_w)], idx_v)

        def gather(j, b):
            pltpu.async_copy(
                table_hbm.at[idx_v.at[j, pl.ds(0, S)]],
                rows_v.at[b, pl.ds(0, S)], gsem.at[b])

        def gather_wait(j, b):
            pltpu.make_async_copy(
                table_hbm.at[idx_v.at[j, pl.ds(0, S)]],
                rows_v.at[b, pl.ds(0, S)], gsem.at[b]).wait()

        def scatter(j, b):
            pltpu.async_copy(
                rows_v.at[pl.ds(b, 1)], out_hbm.at[pl.ds(row0 + j, 1)],
                ssem.at[b])

        def scatter_wait(j, b):
            pltpu.make_async_copy(
                rows_v.at[pl.ds(b, 1)], out_hbm.at[pl.ds(row0 + j, 1)],
                ssem.at[b]).wait()

        for p in range(lead):
            gather(p, p)

        @pl.loop(0, rows_per_w)
        def _(j):
            b = lax.rem(j, nbuf)
            jn = j + lead
            bn = lax.rem(jn, nbuf)

            @pl.when(jn < rows_per_w)
            def _():
                @pl.when(jn >= nbuf)
                def _():
                    scatter_wait(jn - nbuf, bn)
                gather(jn, bn)

            gather_wait(j, b)
            scatter(j, b)

        for t in range(nbuf):
            j = rows_per_w - nbuf + t
            scatter_wait(j, j % nbuf)

    return pl.kernel(
        body,
        out_type=jax.ShapeDtypeStruct((B, SP, DP), jnp.float32),
        mesh=mesh,
        scratch_types=[
            pltpu.VMEM((rows_per_w, DP), jnp.int32),
            pltpu.VMEM((nbuf, SP, DP), jnp.float32),
            pltpu.SemaphoreType.DMA((nbuf,)),
            pltpu.SemaphoreType.DMA((nbuf,)),
        ],
        compiler_params=pltpu.CompilerParams(use_tc_tiling_on_sc=True),
    ), nw


def kernel(token_ids, emb):
    lookup, nw = _build_lookup()
    table = jnp.pad(emb, ((0, 0), (0, DP - D)))
    ids = jnp.pad(token_ids, ((0, 0), (0, DP - S)))
    return lookup(ids, table)[:, :S, :D]


# speedup vs baseline: 1.0053x; 1.0053x over previous
import functools

import jax
import jax.numpy as jnp
from jax import lax
from jax.experimental import pallas as pl
from jax.experimental.pallas import tpu as pltpu
from jax.experimental.pallas import tpu_sc as plsc

B = 4096
S = 50
D = 64
SP = 56
DP = 128


@functools.cache
def _build_lookup():
    info = plsc.get_sparse_core_info()
    nc, ns = info.num_cores, info.num_subcores
    nw = nc * ns
    rows_per_w = B // nw
    nbuf = 8
    lead = 4

    mesh = plsc.VectorSubcoreMesh(core_axis_name="c", subcore_axis_name="s")

    def body(idx_hbm, table_hbm, out_hbm, idx_v, rows_v, gsem, ssem):
        wid = lax.axis_index("s") * nc + lax.axis_index("c")
        row0 = wid * rows_per_w
        pltpu.sync_copy(idx_hbm.at[pl.ds(row0, rows_per_w)], idx_v)

        def gather(j, b):
            pltpu.async_copy(
                table_hbm.at[idx_v.at[j]], rows_v.at[b, pl.ds(0, S)],
                gsem.at[b])

        def gather_wait(j, b):
            pltpu.make_async_copy(
                table_hbm.at[idx_v.at[j]], rows_v.at[b, pl.ds(0, S)],
                gsem.at[b]).wait()

        def scatter(j, b):
            pltpu.async_copy(
                rows_v.at[pl.ds(b, 1)], out_hbm.at[pl.ds(row0 + j, 1)],
                ssem.at[b])

        def scatter_wait(j, b):
            pltpu.make_async_copy(
                rows_v.at[pl.ds(b, 1)], out_hbm.at[pl.ds(row0 + j, 1)],
                ssem.at[b]).wait()

        for p in range(lead):
            gather(p, p)

        @pl.loop(0, rows_per_w)
        def _(j):
            b = lax.rem(j, nbuf)
            jn = j + lead
            bn = lax.rem(jn, nbuf)

            @pl.when(jn < rows_per_w)
            def _():
                @pl.when(jn >= nbuf)
                def _():
                    scatter_wait(jn - nbuf, bn)
                gather(jn, bn)

            gather_wait(j, b)
            scatter(j, b)

        for t in range(nbuf):
            j = rows_per_w - nbuf + t
            scatter_wait(j, j % nbuf)

    return pl.kernel(
        body,
        out_type=jax.ShapeDtypeStruct((B, SP, DP), jnp.float32),
        mesh=mesh,
        scratch_types=[
            pltpu.VMEM((rows_per_w, S), jnp.int32),
            pltpu.VMEM((nbuf, SP, DP), jnp.float32),
            pltpu.SemaphoreType.DMA((nbuf,)),
            pltpu.SemaphoreType.DMA((nbuf,)),
        ],
        compiler_params=pltpu.CompilerParams(use_tc_tiling_on_sc=True),
    ), nw


def kernel(token_ids, emb):
    lookup, nw = _build_lookup()
    table = jnp.pad(emb, ((0, 0), (0, DP - D)))
    return lookup(token_ids, table)[:, :S, :D]
